# Initial kernel scaffold; baseline (speedup 1.0000x reference)
import jax
import jax.numpy as jnp
from jax.experimental import pallas as pl


def kernel(boxes, scores, idxs):
    raise NotImplementedError("write your pallas kernel here")


# speedup vs baseline: 138.1287x; 138.1287x over previous
import functools

import jax
import jax.numpy as jnp
from jax.experimental import pallas as pl
from jax.experimental.pallas import tpu as pltpu

_B = 512
_THR = 0.5


def _iou_overlap(colb, x1j, y1j, x2j, y2j, aj):
    bb = (_B, _B)
    xx1 = jnp.maximum(jnp.broadcast_to(colb["x1"], bb), jnp.broadcast_to(x1j, bb))
    yy1 = jnp.maximum(jnp.broadcast_to(colb["y1"], bb), jnp.broadcast_to(y1j, bb))
    xx2 = jnp.minimum(jnp.broadcast_to(colb["x2"], bb), jnp.broadcast_to(x2j, bb))
    yy2 = jnp.minimum(jnp.broadcast_to(colb["y2"], bb), jnp.broadcast_to(y2j, bb))
    w = jnp.maximum(xx2 - xx1, 0.0)
    h = jnp.maximum(yy2 - yy1, 0.0)
    inter = w * h
    iou = inter / (jnp.broadcast_to(colb["a"], bb) + jnp.broadcast_to(aj, bb)
                   - inter + 1e-9)
    return jnp.where(iou > _THR, 1.0, 0.0)


def _nms_body(nb, jend_ref, colarr_ref, x1r_ref, y1r_ref, x2r_ref, y2r_ref,
              bT_ref, sT_ref, outT_ref, alive_ref):
    alive_ref[...] = jnp.ones((nb, _B), jnp.float32)
    upper = (jax.lax.broadcasted_iota(jnp.int32, (_B, _B), 0)
             < jax.lax.broadcasted_iota(jnp.int32, (_B, _B), 1))
    upper_f = jnp.where(upper, 1.0, 0.0)

    def row_block(bj):
        x1 = x1r_ref[pl.ds(bj, 1), :]
        y1 = y1r_ref[pl.ds(bj, 1), :]
        x2 = x2r_ref[pl.ds(bj, 1), :]
        y2 = y2r_ref[pl.ds(bj, 1), :]
        a = (x2 - x1) * (y2 - y1)
        return x1, y1, x2, y2, a

    def outer(bi, carry):
        base = bi * _B
        cols = colarr_ref[pl.ds(base, _B), :]
        cx1 = cols[:, 0:1]
        cy1 = cols[:, 1:2]
        cx2 = cols[:, 2:3]
        cy2 = cols[:, 3:4]
        colb = {"x1": cx1, "y1": cy1, "x2": cx2, "y2": cy2,
                "a": (cx2 - cx1) * (cy2 - cy1)}

        x1i, y1i, x2i, y2i, ai = row_block(bi)
        omat = _iou_overlap(colb, x1i, y1i, x2i, y2i, ai) * upper_f

        alive_i = alive_ref[pl.ds(bi, 1), :]

        def fp_cond(c):
            return c[1]

        def fp_body(c):
            k, _ = c
            sup = jax.lax.dot_general(
                k, omat, (((1,), (0,)), ((), ())),
                preferred_element_type=jnp.float32)
            knew = jnp.where(sup > 0.0, 0.0, alive_i)
            return knew, jnp.any(knew != k)

        keep, _ = jax.lax.while_loop(fp_cond, fp_body,
                                     (alive_i, jnp.asarray(True)))

        sl = pl.ds(base, _B)
        out5 = jnp.concatenate(
            [bT_ref[:, sl] * keep, sT_ref[:, sl] * keep], axis=0)
        outT_ref[:, sl] = out5

        def inner(bj, c2):
            x1j, y1j, x2j, y2j, aj = row_block(bj)
            ocross = _iou_overlap(colb, x1j, y1j, x2j, y2j, aj)
            sup = jax.lax.dot_general(
                keep, ocross, (((1,), (0,)), ((), ())),
                preferred_element_type=jnp.float32)
            row = alive_ref[pl.ds(bj, 1), :]
            alive_ref[pl.ds(bj, 1), :] = jnp.where(sup > 0.0, 0.0, row)
            return c2

        jax.lax.fori_loop(bi + 1, jend_ref[bi], inner, 0)
        return carry

    jax.lax.fori_loop(0, nb, outer, 0)


@functools.partial(jax.jit, static_argnums=())
def kernel(boxes, scores, idxs):
    n = boxes.shape[0]
    npad = ((n + _B - 1) // _B) * _B
    nb = npad // _B

    max_coordinate = jnp.max(boxes) + 1.0
    offsets = idxs.astype(boxes.dtype) * max_coordinate
    boxes_nms = boxes + offsets[:, None]

    order = jnp.argsort(-scores)
    bn = boxes_nms[order]
    bs = boxes[order]
    ss = scores[order]

    pad = npad - n
    bn_p = jnp.concatenate(
        [bn, jnp.full((pad, 4), -1e6, jnp.float32)], axis=0)
    bs_p = jnp.concatenate([bs, jnp.zeros((pad, 4), jnp.float32)], axis=0)
    ss_p = jnp.concatenate([ss, jnp.zeros((pad,), jnp.float32)], axis=0)

    x1r = bn_p[:, 0].reshape(nb, _B)
    y1r = bn_p[:, 1].reshape(nb, _B)
    x2r = bn_p[:, 2].reshape(nb, _B)
    y2r = bn_p[:, 3].reshape(nb, _B)
    bT = bs_p.T
    sT = ss_p.reshape(1, npad)
    jend = jnp.full((nb,), nb, jnp.int32)

    outT = pl.pallas_call(
        functools.partial(_nms_body, nb),
        out_shape=jax.ShapeDtypeStruct((5, npad), jnp.float32),
        in_specs=[
            pl.BlockSpec(memory_space=pltpu.SMEM),
            pl.BlockSpec(memory_space=pltpu.VMEM),
            pl.BlockSpec(memory_space=pltpu.VMEM),
            pl.BlockSpec(memory_space=pltpu.VMEM),
            pl.BlockSpec(memory_space=pltpu.VMEM),
            pl.BlockSpec(memory_space=pltpu.VMEM),
            pl.BlockSpec(memory_space=pltpu.VMEM),
            pl.BlockSpec(memory_space=pltpu.VMEM),
        ],
        out_specs=pl.BlockSpec(memory_space=pltpu.VMEM),
        scratch_shapes=[pltpu.VMEM((nb, _B), jnp.float32)],
    )(jend, bn_p, x1r, y1r, x2r, y2r, bT, sT)

    return outT[:, :n].T
